# baseline (speedup 1.0000x reference)
import functools

import jax
import jax.numpy as jnp
from jax import lax
from jax.experimental import pallas as pl
from jax.experimental.pallas import tpu as pltpu
from jax.experimental.pallas import tpu_sc as plsc

B = 4096
L = 200
D = 32
NC = 2
NS = 16
NW = NC * NS
BB = B // 128
LA = L // 8
PITCH = 133

_mesh = plsc.VectorSubcoreMesh(core_axis_name="c", subcore_axis_name="s")

V = 1000000
VB = 512
VFULL = V // VB
VTAIL = V - VFULL * VB
ABLK = 62


@functools.partial(
    pl.kernel,
    out_type=jax.ShapeDtypeStruct((V // 4, 128), jnp.float32),
    mesh=_mesh,
    scratch_types=[
        pltpu.VMEM((D, VB + 1), jnp.float32),
        pltpu.VMEM((D, VB + 1), jnp.float32),
        pltpu.VMEM((VB // 4, 136), jnp.float32),
        pltpu.VMEM((VB // 4, 136), jnp.float32),
        pltpu.SemaphoreType.DMA,
        pltpu.SemaphoreType.DMA,
        pltpu.SemaphoreType.DMA,
        pltpu.SemaphoreType.DMA,
    ],
    compiler_params=pltpu.CompilerParams(
        needs_layout_passes=False,
        disable_bounds_checks=True,
        disable_semaphore_checks=True,
    ),
)
def _sc_table_rowmajor(wt2_hbm, tail_hbm, wtr_hbm,
                       inb0, inb1, tpad0, tpad1,
                       gsem0, gsem1, ssem0, ssem1):
    w = lax.axis_index("s") * NC + lax.axis_index("c")

    @pl.when(w == 31)
    def _():
        pltpu.sync_copy(tail_hbm, wtr_hbm.at[pl.ds(VFULL * (VB // 4), 16)])

    inbs = [inb0, inb1]
    tpads = [tpad0, tpad1]
    gsems = [gsem0, gsem1]
    ssems = [ssem0, ssem1]
    iota = jnp.arange(16, dtype=jnp.int32)
    d_lo = iota
    d_hi = iota + 16

    def drain_stores(p):
        pltpu.make_async_copy(
            wtr_hbm.at[pl.ds(0, VB // 4)],
            tpads[p].at[:, pl.ds(0, 128)],
            ssems[p],
        ).wait()

    def fire(k, q):
        blk = k * NW + w

        @pl.when(blk < VFULL)
        def _():
            col = pl.multiple_of(blk * VB, 128)
            pltpu.async_copy(
                wt2_hbm.at[:, pl.ds(col, VB)], inbs[q].at[:, pl.ds(0, VB)], gsems[q]
            )

    fire(jnp.int32(0), 0)

    def step(k, p):
        q = p ^ 1
        fire(k + 1, q)
        blk = k * NW + w

        @pl.when((k >= 2) & ((k - 2) * NW + w < VFULL))
        def _():
            drain_stores(p)

        @pl.when(blk < VFULL)
        def _():
            pltpu.make_async_copy(
                wt2_hbm.at[:, pl.ds(0, VB)], inbs[p].at[:, pl.ds(0, VB)], gsems[p]
            ).wait()

            inb, tpad = inbs[p], tpads[p]

            @plsc.parallel_loop(0, VB, unroll=4)
            def _(s):
                colv = jnp.broadcast_to(s, (16,))
                r = s >> 2
                c = (s & 3) * 32
                tpad[r, pl.ds(c, 16)] = plsc.load_gather(inb, [d_lo, colv])
                tpad[r, pl.ds(c + 16, 16)] = plsc.load_gather(inb, [d_hi, colv])

            pltpu.async_copy(
                tpad.at[:, pl.ds(0, 128)],
                wtr_hbm.at[pl.ds(blk * (VB // 4), VB // 4)],
                ssems[p],
            )

    def body(j, _):
        step(2 * j, 0)
        step(2 * j + 1, 1)
        return 0

    lax.fori_loop(0, ABLK // 2, body, 0)
    @pl.when((ABLK - 2) * NW + w < VFULL)
    def _():
        drain_stores(0)


@functools.partial(
    pl.kernel,
    out_type=jax.ShapeDtypeStruct((L, D // 8, BB, 8, 128), jnp.float32),
    mesh=_mesh,
    scratch_types=[
        pltpu.VMEM((LA, 8, 128), jnp.int32),
        pltpu.VMEM((128, D), jnp.float32),
        pltpu.VMEM((128, D), jnp.float32),
        pltpu.VMEM((128, D), jnp.float32),
        pltpu.VMEM((128, D), jnp.float32),
        pltpu.VMEM((D // 8, 8, PITCH), jnp.float32),
        pltpu.VMEM((D // 8, 8, PITCH), jnp.float32),
        pltpu.VMEM((L, D), jnp.float32),
        pltpu.SemaphoreType.DMA,
        pltpu.SemaphoreType.DMA,
        pltpu.SemaphoreType.DMA,
        pltpu.SemaphoreType.DMA,
        pltpu.SemaphoreType.DMA,
        pltpu.SemaphoreType.DMA,
    ],
    compiler_params=pltpu.CompilerParams(
        use_tc_tiling_on_sc=False,
        needs_layout_passes=False,
        disable_bounds_checks=True,
        disable_semaphore_checks=True,
    ),
)
def _sc_embed(idx_hbm, table_hbm, pos_hbm, out_hbm,
              idxcol, gbuf0, gbuf1, gbuf2, gbuf3, sout0, sout1, posv,
              gsem0, gsem1, gsem2, gsem3, ssem0, ssem1):
    w = lax.axis_index("s") * NC + lax.axis_index("c")
    pltpu.sync_copy(pos_hbm, posv)
    pltpu.sync_copy(idx_hbm.at[:, w], idxcol)

    gbufs = [gbuf0, gbuf1, gbuf2, gbuf3]
    souts = [sout0, sout1]
    gsems = [gsem0, gsem1, gsem2, gsem3]
    ssems = [ssem0, ssem1]
    iota = jnp.arange(16, dtype=jnp.int32)
    r_lo, u_lo = iota // 8, iota % 8
    r_hi, u_hi = (iota + 16) // 8, iota % 8

    def fire_gather(l, q):
        pltpu.async_copy(
            table_hbm.at[idxcol.at[l >> 3, l & 7]], gbufs[q], gsems[q]
        )

    for l0 in range(3):
        fire_gather(jnp.int32(l0), l0)

    def step(l, t):
        nxt = l + 3

        @pl.when(nxt < L)
        def _():
            fire_gather(nxt, (t + 3) % 4)

        pltpu.make_async_copy(table_hbm.at[idxcol.at[0, 0]], gbufs[t], gsems[t]).wait()

        sp = t % 2

        @pl.when(l >= 2)
        def _():
            pltpu.make_async_copy(
                out_hbm.at[0, :, 0], souts[sp].at[:, :, pl.ds(0, 128)], ssems[sp]
            ).wait()

        gb, so = gbufs[t], souts[sp]
        pos_lo = posv[l, pl.ds(0, 16)]
        pos_hi = posv[l, pl.ds(16, 16)]

        @plsc.parallel_loop(0, 128, unroll=8)
        def _(b):
            col = jnp.broadcast_to(b, (16,))
            plsc.store_scatter(so, [r_lo, u_lo, col], gb[b, pl.ds(0, 16)] + pos_lo)
            plsc.store_scatter(so, [r_hi, u_hi, col], gb[b, pl.ds(16, 16)] + pos_hi)
        pltpu.async_copy(
            so.at[:, :, pl.ds(0, 128)], out_hbm.at[l, :, w], ssems[sp]
        )

    def body(j, _):
        for t in range(4):
            step(4 * j + t, t)
        return 0

    lax.fori_loop(0, L // 4, body, 0)
    pltpu.make_async_copy(out_hbm.at[0, :, 0], sout0.at[:, :, pl.ds(0, 128)], ssem0).wait()
    pltpu.make_async_copy(out_hbm.at[0, :, 0], sout1.at[:, :, pl.ds(0, 128)], ssem1).wait()


def kernel(inputs, word_table, pos_table):
    idx4 = (
        inputs.T.astype(jnp.int32)
        .reshape(LA, 8, BB, 128)
        .transpose(0, 2, 1, 3)
    )
    tail = word_table[VFULL * VB :].reshape(16, 128)
    wtr = _sc_table_rowmajor(word_table.T, tail)
    x = _sc_embed(idx4, wtr.reshape(V, D), pos_table)
    return x.transpose(2, 4, 0, 1, 3).reshape(B, L, D)


# speedup vs baseline: 1.0061x; 1.0061x over previous
import functools

import jax
import jax.numpy as jnp
from jax import lax
from jax.experimental import pallas as pl
from jax.experimental.pallas import tpu as pltpu
from jax.experimental.pallas import tpu_sc as plsc

B = 4096
L = 200
D = 32
NC = 2
NS = 16
NW = NC * NS
BB = B // 128
LA = L // 8
PITCH = 133

_mesh = plsc.VectorSubcoreMesh(core_axis_name="c", subcore_axis_name="s")

V = 1000000
VB = 256
VFULL = V // VB
VTAIL = V - VFULL * VB
ABLK = 124


@functools.partial(
    pl.kernel,
    out_type=jax.ShapeDtypeStruct((V // 4, 128), jnp.float32),
    mesh=_mesh,
    scratch_types=[
        pltpu.VMEM((D, VB + 1), jnp.float32),
        pltpu.VMEM((D, VB + 1), jnp.float32),
        pltpu.VMEM((VB // 4, 136), jnp.float32),
        pltpu.VMEM((VB // 4, 136), jnp.float32),
        pltpu.SemaphoreType.DMA,
        pltpu.SemaphoreType.DMA,
        pltpu.SemaphoreType.DMA,
        pltpu.SemaphoreType.DMA,
    ],
    compiler_params=pltpu.CompilerParams(
        needs_layout_passes=False,
        disable_bounds_checks=True,
        disable_semaphore_checks=True,
    ),
)
def _sc_table_rowmajor(wt2_hbm, tail_hbm, wtr_hbm,
                       inb0, inb1, tpad0, tpad1,
                       gsem0, gsem1, ssem0, ssem1):
    w = lax.axis_index("s") * NC + lax.axis_index("c")

    @pl.when(w == 31)
    def _():
        pltpu.sync_copy(tail_hbm, wtr_hbm.at[pl.ds(VFULL * (VB // 4), 16)])

    inbs = [inb0, inb1]
    tpads = [tpad0, tpad1]
    gsems = [gsem0, gsem1]
    ssems = [ssem0, ssem1]
    iota = jnp.arange(16, dtype=jnp.int32)
    d_lo = iota
    d_hi = iota + 16

    def drain_stores(p):
        pltpu.make_async_copy(
            wtr_hbm.at[pl.ds(0, VB // 4)],
            tpads[p].at[:, pl.ds(0, 128)],
            ssems[p],
        ).wait()

    def fire(k, q):
        blk = k * NW + w

        @pl.when(blk < VFULL)
        def _():
            col = pl.multiple_of(blk * VB, 128)
            pltpu.async_copy(
                wt2_hbm.at[:, pl.ds(col, VB)], inbs[q].at[:, pl.ds(0, VB)], gsems[q]
            )

    fire(jnp.int32(0), 0)

    def step(k, p):
        q = p ^ 1
        fire(k + 1, q)
        blk = k * NW + w

        @pl.when((k >= 2) & ((k - 2) * NW + w < VFULL))
        def _():
            drain_stores(p)

        @pl.when(blk < VFULL)
        def _():
            pltpu.make_async_copy(
                wt2_hbm.at[:, pl.ds(0, VB)], inbs[p].at[:, pl.ds(0, VB)], gsems[p]
            ).wait()

            inb, tpad = inbs[p], tpads[p]

            @plsc.parallel_loop(0, VB, unroll=4)
            def _(s):
                colv = jnp.broadcast_to(s, (16,))
                r = s >> 2
                c = (s & 3) * 32
                tpad[r, pl.ds(c, 16)] = plsc.load_gather(inb, [d_lo, colv])
                tpad[r, pl.ds(c + 16, 16)] = plsc.load_gather(inb, [d_hi, colv])

            pltpu.async_copy(
                tpad.at[:, pl.ds(0, 128)],
                wtr_hbm.at[pl.ds(blk * (VB // 4), VB // 4)],
                ssems[p],
            )

    def body(j, _):
        step(2 * j, 0)
        step(2 * j + 1, 1)
        return 0

    lax.fori_loop(0, ABLK // 2, body, 0)
    @pl.when((ABLK - 2) * NW + w < VFULL)
    def _():
        drain_stores(0)

    @pl.when((ABLK - 1) * NW + w < VFULL)
    def _():
        drain_stores(1)


@functools.partial(
    pl.kernel,
    out_type=jax.ShapeDtypeStruct((L, D // 8, BB, 8, 128), jnp.float32),
    mesh=_mesh,
    scratch_types=[
        pltpu.VMEM((LA, 8, 128), jnp.int32),
        pltpu.VMEM((128, D), jnp.float32),
        pltpu.VMEM((128, D), jnp.float32),
        pltpu.VMEM((128, D), jnp.float32),
        pltpu.VMEM((128, D), jnp.float32),
        pltpu.VMEM((D // 8, 8, PITCH), jnp.float32),
        pltpu.VMEM((D // 8, 8, PITCH), jnp.float32),
        pltpu.VMEM((L, D), jnp.float32),
        pltpu.SemaphoreType.DMA,
        pltpu.SemaphoreType.DMA,
        pltpu.SemaphoreType.DMA,
        pltpu.SemaphoreType.DMA,
        pltpu.SemaphoreType.DMA,
        pltpu.SemaphoreType.DMA,
    ],
    compiler_params=pltpu.CompilerParams(
        use_tc_tiling_on_sc=False,
        needs_layout_passes=False,
        disable_bounds_checks=True,
        disable_semaphore_checks=True,
    ),
)
def _sc_embed(idx_hbm, table_hbm, pos_hbm, out_hbm,
              idxcol, gbuf0, gbuf1, gbuf2, gbuf3, sout0, sout1, posv,
              gsem0, gsem1, gsem2, gsem3, ssem0, ssem1):
    w = lax.axis_index("s") * NC + lax.axis_index("c")
    pltpu.sync_copy(pos_hbm, posv)
    pltpu.sync_copy(idx_hbm.at[:, w], idxcol)

    gbufs = [gbuf0, gbuf1, gbuf2, gbuf3]
    souts = [sout0, sout1]
    gsems = [gsem0, gsem1, gsem2, gsem3]
    ssems = [ssem0, ssem1]
    iota = jnp.arange(16, dtype=jnp.int32)
    r_lo, u_lo = iota // 8, iota % 8
    r_hi, u_hi = (iota + 16) // 8, iota % 8

    def fire_gather(l, q):
        pltpu.async_copy(
            table_hbm.at[idxcol.at[l >> 3, l & 7]], gbufs[q], gsems[q]
        )

    for l0 in range(3):
        fire_gather(jnp.int32(l0), l0)

    def step(l, t):
        nxt = l + 3

        @pl.when(nxt < L)
        def _():
            fire_gather(nxt, (t + 3) % 4)

        pltpu.make_async_copy(table_hbm.at[idxcol.at[0, 0]], gbufs[t], gsems[t]).wait()

        sp = t % 2

        @pl.when(l >= 2)
        def _():
            pltpu.make_async_copy(
                out_hbm.at[0, :, 0], souts[sp].at[:, :, pl.ds(0, 128)], ssems[sp]
            ).wait()

        gb, so = gbufs[t], souts[sp]
        pos_lo = posv[l, pl.ds(0, 16)]
        pos_hi = posv[l, pl.ds(16, 16)]

        @plsc.parallel_loop(0, 128, unroll=8)
        def _(b):
            col = jnp.broadcast_to(b, (16,))
            plsc.store_scatter(so, [r_lo, u_lo, col], gb[b, pl.ds(0, 16)] + pos_lo)
            plsc.store_scatter(so, [r_hi, u_hi, col], gb[b, pl.ds(16, 16)] + pos_hi)
        pltpu.async_copy(
            so.at[:, :, pl.ds(0, 128)], out_hbm.at[l, :, w], ssems[sp]
        )

    def body(j, _):
        for t in range(4):
            step(4 * j + t, t)
        return 0

    lax.fori_loop(0, L // 4, body, 0)
    pltpu.make_async_copy(out_hbm.at[0, :, 0], sout0.at[:, :, pl.ds(0, 128)], ssem0).wait()
    pltpu.make_async_copy(out_hbm.at[0, :, 0], sout1.at[:, :, pl.ds(0, 128)], ssem1).wait()


def kernel(inputs, word_table, pos_table):
    idx4 = (
        inputs.T.astype(jnp.int32)
        .reshape(LA, 8, BB, 128)
        .transpose(0, 2, 1, 3)
    )
    tail = word_table[VFULL * VB :].reshape(16, 128)
    wtr = _sc_table_rowmajor(word_table.T, tail)
    x = _sc_embed(idx4, wtr.reshape(V, D), pos_table)
    return x.transpose(2, 4, 0, 1, 3).reshape(B, L, D)


# speedup vs baseline: 1.0088x; 1.0027x over previous
import functools

import jax
import jax.numpy as jnp
from jax import lax
from jax.experimental import pallas as pl
from jax.experimental.pallas import tpu as pltpu
from jax.experimental.pallas import tpu_sc as plsc

B = 4096
L = 200
D = 32
NC = 2
NS = 16
NW = NC * NS
BB = B // 128
LA = L // 8
PITCH = 133

_mesh = plsc.VectorSubcoreMesh(core_axis_name="c", subcore_axis_name="s")

V = 1000000
VB = 256
VFULL = V // VB
VTAIL = V - VFULL * VB
ABLK = 124


@functools.partial(
    pl.kernel,
    out_type=jax.ShapeDtypeStruct((V // 4, 128), jnp.float32),
    mesh=_mesh,
    scratch_types=[
        pltpu.VMEM((D, VB + 1), jnp.float32),
        pltpu.VMEM((D, VB + 1), jnp.float32),
        pltpu.VMEM((VB // 4, 136), jnp.float32),
        pltpu.VMEM((VB // 4, 136), jnp.float32),
        pltpu.SemaphoreType.DMA,
        pltpu.SemaphoreType.DMA,
        pltpu.SemaphoreType.DMA,
        pltpu.SemaphoreType.DMA,
    ],
    compiler_params=pltpu.CompilerParams(
        needs_layout_passes=False,
        disable_bounds_checks=True,
        disable_semaphore_checks=True,
    ),
)
def _sc_table_rowmajor(wt2_hbm, tail_hbm, wtr_hbm,
                       inb0, inb1, tpad0, tpad1,
                       gsem0, gsem1, ssem0, ssem1):
    w = lax.axis_index("s") * NC + lax.axis_index("c")

    @pl.when(w == 31)
    def _():
        pltpu.sync_copy(tail_hbm, wtr_hbm.at[pl.ds(VFULL * (VB // 4), 16)])

    inbs = [inb0, inb1]
    tpads = [tpad0, tpad1]
    gsems = [gsem0, gsem1]
    ssems = [ssem0, ssem1]
    iota = jnp.arange(16, dtype=jnp.int32)
    d_lo = iota
    d_hi = iota + 16

    def drain_stores(p):
        pltpu.make_async_copy(
            wtr_hbm.at[pl.ds(0, VB // 4)],
            tpads[p].at[:, pl.ds(0, 128)],
            ssems[p],
        ).wait()

    def fire(k, q):
        blk = k * NW + w

        @pl.when(blk < VFULL)
        def _():
            col = pl.multiple_of(blk * VB, 128)
            pltpu.async_copy(
                wt2_hbm.at[:, pl.ds(col, VB)], inbs[q].at[:, pl.ds(0, VB)], gsems[q]
            )

    fire(jnp.int32(0), 0)

    def step(k, p):
        q = p ^ 1
        fire(k + 1, q)
        blk = k * NW + w

        @pl.when((k >= 2) & ((k - 2) * NW + w < VFULL))
        def _():
            drain_stores(p)

        @pl.when(blk < VFULL)
        def _():
            pltpu.make_async_copy(
                wt2_hbm.at[:, pl.ds(0, VB)], inbs[p].at[:, pl.ds(0, VB)], gsems[p]
            ).wait()

            inb, tpad = inbs[p], tpads[p]

            @plsc.parallel_loop(0, VB, unroll=8)
            def _(s):
                colv = jnp.broadcast_to(s, (16,))
                r = s >> 2
                c = (s & 3) * 32
                tpad[r, pl.ds(c, 16)] = plsc.load_gather(inb, [d_lo, colv])
                tpad[r, pl.ds(c + 16, 16)] = plsc.load_gather(inb, [d_hi, colv])

            pltpu.async_copy(
                tpad.at[:, pl.ds(0, 128)],
                wtr_hbm.at[pl.ds(blk * (VB // 4), VB // 4)],
                ssems[p],
            )

    def body(j, _):
        step(2 * j, 0)
        step(2 * j + 1, 1)
        return 0

    lax.fori_loop(0, ABLK // 2, body, 0)
    @pl.when((ABLK - 2) * NW + w < VFULL)
    def _():
        drain_stores(0)

    @pl.when((ABLK - 1) * NW + w < VFULL)
    def _():
        drain_stores(1)


@functools.partial(
    pl.kernel,
    out_type=jax.ShapeDtypeStruct((L, D // 8, BB, 8, 128), jnp.float32),
    mesh=_mesh,
    scratch_types=[
        pltpu.VMEM((LA, 8, 128), jnp.int32),
        pltpu.VMEM((128, D), jnp.float32),
        pltpu.VMEM((128, D), jnp.float32),
        pltpu.VMEM((128, D), jnp.float32),
        pltpu.VMEM((128, D), jnp.float32),
        pltpu.VMEM((D // 8, 8, PITCH), jnp.float32),
        pltpu.VMEM((D // 8, 8, PITCH), jnp.float32),
        pltpu.VMEM((L, D), jnp.float32),
        pltpu.SemaphoreType.DMA,
        pltpu.SemaphoreType.DMA,
        pltpu.SemaphoreType.DMA,
        pltpu.SemaphoreType.DMA,
        pltpu.SemaphoreType.DMA,
        pltpu.SemaphoreType.DMA,
    ],
    compiler_params=pltpu.CompilerParams(
        use_tc_tiling_on_sc=False,
        needs_layout_passes=False,
        disable_bounds_checks=True,
        disable_semaphore_checks=True,
    ),
)
def _sc_embed(idx_hbm, table_hbm, pos_hbm, out_hbm,
              idxcol, gbuf0, gbuf1, gbuf2, gbuf3, sout0, sout1, posv,
              gsem0, gsem1, gsem2, gsem3, ssem0, ssem1):
    w = lax.axis_index("s") * NC + lax.axis_index("c")
    pltpu.sync_copy(pos_hbm, posv)
    pltpu.sync_copy(idx_hbm.at[:, w], idxcol)

    gbufs = [gbuf0, gbuf1, gbuf2, gbuf3]
    souts = [sout0, sout1]
    gsems = [gsem0, gsem1, gsem2, gsem3]
    ssems = [ssem0, ssem1]
    iota = jnp.arange(16, dtype=jnp.int32)
    r_lo, u_lo = iota // 8, iota % 8
    r_hi, u_hi = (iota + 16) // 8, iota % 8

    def fire_gather(l, q):
        pltpu.async_copy(
            table_hbm.at[idxcol.at[l >> 3, l & 7]], gbufs[q], gsems[q]
        )

    for l0 in range(3):
        fire_gather(jnp.int32(l0), l0)

    def step(l, t):
        nxt = l + 3

        @pl.when(nxt < L)
        def _():
            fire_gather(nxt, (t + 3) % 4)

        pltpu.make_async_copy(table_hbm.at[idxcol.at[0, 0]], gbufs[t], gsems[t]).wait()

        sp = t % 2

        @pl.when(l >= 2)
        def _():
            pltpu.make_async_copy(
                out_hbm.at[0, :, 0], souts[sp].at[:, :, pl.ds(0, 128)], ssems[sp]
            ).wait()

        gb, so = gbufs[t], souts[sp]
        pos_lo = posv[l, pl.ds(0, 16)]
        pos_hi = posv[l, pl.ds(16, 16)]

        @plsc.parallel_loop(0, 128, unroll=8)
        def _(b):
            col = jnp.broadcast_to(b, (16,))
            plsc.store_scatter(so, [r_lo, u_lo, col], gb[b, pl.ds(0, 16)] + pos_lo)
            plsc.store_scatter(so, [r_hi, u_hi, col], gb[b, pl.ds(16, 16)] + pos_hi)
        pltpu.async_copy(
            so.at[:, :, pl.ds(0, 128)], out_hbm.at[l, :, w], ssems[sp]
        )

    def body(j, _):
        for t in range(4):
            step(4 * j + t, t)
        return 0

    lax.fori_loop(0, L // 4, body, 0)
    pltpu.make_async_copy(out_hbm.at[0, :, 0], sout0.at[:, :, pl.ds(0, 128)], ssem0).wait()
    pltpu.make_async_copy(out_hbm.at[0, :, 0], sout1.at[:, :, pl.ds(0, 128)], ssem1).wait()


def kernel(inputs, word_table, pos_table):
    idx4 = (
        inputs.T.astype(jnp.int32)
        .reshape(LA, 8, BB, 128)
        .transpose(0, 2, 1, 3)
    )
    tail = word_table[VFULL * VB :].reshape(16, 128)
    wtr = _sc_table_rowmajor(word_table.T, tail)
    x = _sc_embed(idx4, wtr.reshape(V, D), pos_table)
    return x.transpose(2, 4, 0, 1, 3).reshape(B, L, D)
